# baseline (speedup 1.0000x reference)
import functools

import jax
import jax.numpy as jnp
from jax import lax
from jax.experimental import pallas as pl
from jax.experimental.pallas import tpu as pltpu
from jax.experimental.pallas import tpu_sc as plsc

BATCH = 4096
F = 26
E = 16
A = 16
VOCAB = 1000000
P = F * (F - 1) // 2
S = (P + 15) // 16
P_PAD = S * 16

NC, NS = 2, 16
NW = NC * NS

TCOLS = 7813
EMB_WORDS = 2 * TCOLS * 8 * 128
FOW_WORDS = TCOLS * 8 * 128

EROWS = F * E
GROWS = 448
BBLK = 256
NBLK = BATCH // BBLK
BPW = 4

SPLIT = 2
BH = BATCH // SPLIT
NBLK_H = BH // BBLK
LBLK = BH // NW


def _sc_body(idx_hbm, emb_flat, fow_flat, out, idx_v, gat_v, sem_e, sem_f):
    wid = lax.axis_index("s") * NC + lax.axis_index("c")
    blk = wid // BPW
    off = (wid % BPW) * LBLK
    pltpu.sync_copy(idx_hbm.at[blk, :, pl.ds(off, LBLK)], idx_v)

    def fire_emb(k, carry):
        pltpu.async_copy(emb_flat.at[idx_v.at[k]], gat_v.at[k], sem_e)
        return carry

    lax.fori_loop(0, EROWS, fire_emb, 0)

    def fire_fow(k, carry):
        pltpu.async_copy(fow_flat.at[idx_v.at[EROWS + k]], gat_v.at[EROWS + k],
                         sem_f)
        return carry

    lax.fori_loop(0, F, fire_fow, 0)

    def drain_emb(k, carry):
        pltpu.make_async_copy(emb_flat.at[idx_v.at[0]], gat_v.at[0],
                              sem_e).wait()
        return carry

    lax.fori_loop(0, EROWS, drain_emb, 0)

    def drain_fow(k, carry):
        pltpu.make_async_copy(fow_flat.at[idx_v.at[0]], gat_v.at[0],
                              sem_f).wait()
        return carry

    lax.fori_loop(0, F, drain_fow, 0)

    pltpu.sync_copy(gat_v, out.at[blk, :, pl.ds(off, LBLK)])


@functools.cache
def _sc_gather_fn():
    mesh = plsc.VectorSubcoreMesh(
        core_axis_name="c", subcore_axis_name="s",
        num_cores=NC, num_subcores=NS)
    return pl.kernel(
        _sc_body,
        out_type=jax.ShapeDtypeStruct((NBLK_H, GROWS, BBLK), jnp.float32),
        mesh=mesh,
        scratch_types=[
            pltpu.VMEM((GROWS, LBLK), jnp.int32),
            pltpu.VMEM((GROWS, LBLK), jnp.float32),
            pltpu.SemaphoreType.DMA,
            pltpu.SemaphoreType.DMA,
        ],
        compiler_params=pltpu.CompilerParams(use_tc_tiling_on_sc=False),
    )


def _tc_body(gat_ref, fvT_ref, bd_ref, bb_ref, hbd_ref,
             pp_ref, bias_ref, out_ref, bi_ref, log_ref):
    fv = fvT_ref[...]
    g = gat_ref[0]
    emb = g[0:EROWS].reshape(F, E, BBLK)
    fow = g[EROWS:EROWS + F]
    ev = emb * fv[:, None, :]

    off = 0
    for i in range(F - 1):
        n = F - 1 - i
        bi_ref[off:off + n] = ev[i + 1:F] * ev[i:i + 1]
        off += n
    bi_ref[P:P_PAD] = jnp.zeros((P_PAD - P, E, BBLK), jnp.float32)

    bd = bd_ref[...]
    bb = bb_ref[...]
    hbd = hbd_ref[...]
    for s in range(S):
        bi_s = bi_ref[s * 16:(s + 1) * 16].reshape(16 * E, BBLK)
        att = jnp.maximum(
            jnp.dot(bd, bi_s, preferred_element_type=jnp.float32) + bb, 0.0)
        log_ref[s * 16:(s + 1) * 16] = jnp.dot(
            hbd, att, preferred_element_type=jnp.float32)
    log_ref[P:P_PAD] = jnp.full((P_PAD - P, BBLK), -1e30, jnp.float32)

    logits = log_ref[...]
    m = jnp.max(logits, axis=0, keepdims=True)
    ex = jnp.exp(logits - m)
    z = jnp.sum(ex, axis=0, keepdims=True)
    score = ex / z

    aw = jnp.sum(score[:, None, :] * bi_ref[...], axis=0)
    awp = jnp.sum(aw * pp_ref[...], axis=0, keepdims=True)
    y1 = jnp.sum(fow * fv, axis=0, keepdims=True)
    y = bias_ref[...] + y1 + awp
    out_ref[...] = (1.0 / (1.0 + jnp.exp(-y)))[None]


def _tc_forward(gat, fvT, bd, bb, hbd, pp, bias_r):
    return pl.pallas_call(
        _tc_body,
        grid=(NBLK_H,),
        in_specs=[
            pl.BlockSpec((1, GROWS, BBLK), lambda i: (i, 0, 0)),
            pl.BlockSpec((F, BBLK), lambda i: (0, i)),
            pl.BlockSpec((16 * A, 16 * E), lambda i: (0, 0)),
            pl.BlockSpec((16 * A, 1), lambda i: (0, 0)),
            pl.BlockSpec((16, 16 * A), lambda i: (0, 0)),
            pl.BlockSpec((E, 1), lambda i: (0, 0)),
            pl.BlockSpec((1, 1), lambda i: (0, 0)),
        ],
        out_specs=pl.BlockSpec((1, 1, BBLK), lambda i: (i, 0, 0)),
        out_shape=jax.ShapeDtypeStruct((NBLK_H, 1, BBLK), jnp.float32),
        scratch_shapes=[
            pltpu.VMEM((P_PAD, E, BBLK), jnp.float32),
            pltpu.VMEM((P_PAD, BBLK), jnp.float32),
        ],
    )(gat, fvT, bd, bb, hbd, pp, bias_r)


def kernel(feat_index, feat_value, first_order_w, emb_table, bias,
           attention_w, attention_b, projection_h, projection_p):
    vT = feat_index.astype(jnp.int32).T
    v3 = vT.reshape(F, NBLK, BBLK).transpose(1, 0, 2)
    er = jnp.arange(E, dtype=jnp.int32)
    e_base = (er // 8) * (TCOLS * 1024) + (er % 8) * 128
    vhi = (v3 >> 7) * 1024 + (v3 & 127)
    emb_idx = (vhi[:, :, None, :] + e_base[None, None, :, None]
               ).reshape(NBLK, EROWS, BBLK)
    big_idx = jnp.concatenate(
        [emb_idx, vhi,
         jnp.zeros((NBLK, GROWS - EROWS - F, BBLK), jnp.int32)], axis=1)

    emb_flat = (jnp.pad(emb_table.T, ((0, 0), (0, 64)))
                .reshape(2, 8, TCOLS, 128).transpose(0, 2, 1, 3)
                .reshape(EMB_WORDS))
    fow_flat = (jnp.pad(first_order_w.T, ((0, 7), (0, 64)))
                .reshape(1, 8, TCOLS, 128).transpose(0, 2, 1, 3)
                .reshape(FOW_WORDS))

    fvT = feat_value.T
    eye = jnp.eye(16, dtype=jnp.float32)
    bd = jnp.kron(eye, attention_w.T)
    bb = jnp.tile(attention_b, 16)[:, None]
    hbd = jnp.kron(eye, projection_h[:, 0][None, :])
    bias_r = bias.reshape(1, 1)

    sc = _sc_gather_fn()
    outs = []
    for h in range(SPLIT):
        gat_h = sc(big_idx[h * NBLK_H:(h + 1) * NBLK_H], emb_flat, fow_flat)
        fvT_h = fvT[:, h * BH:(h + 1) * BH]
        outs.append(_tc_forward(gat_h, fvT_h, bd, bb, hbd,
                                projection_p, bias_r))
    return jnp.concatenate(outs, axis=0).reshape(BATCH)


# speedup vs baseline: 1.1331x; 1.1331x over previous
import functools

import jax
import jax.numpy as jnp
from jax import lax
from jax.experimental import pallas as pl
from jax.experimental.pallas import tpu as pltpu
from jax.experimental.pallas import tpu_sc as plsc

BATCH = 4096
F = 26
E = 16
A = 16
VOCAB = 1000000
P = F * (F - 1) // 2
S = (P + 15) // 16
P_PAD = S * 16

NC, NS = 2, 16
NW = NC * NS

TCOLS = 7813
EMB_WORDS = 2 * TCOLS * 8 * 128
FOW_WORDS = TCOLS * 8 * 128

EROWS = F * E
GROWS = 448
BBLK = 128
NBLK = BATCH // BBLK
BPW = 4

SPLIT = 4
BH = BATCH // SPLIT
NBLK_H = BH // BBLK
LBLK = BH // NW


def _sc_body(idx_hbm, emb_flat, fow_flat, out, idx_v, gat_v, sem_e, sem_f):
    wid = lax.axis_index("s") * NC + lax.axis_index("c")
    blk = wid // BPW
    off = (wid % BPW) * LBLK
    pltpu.sync_copy(idx_hbm.at[blk, :, pl.ds(off, LBLK)], idx_v)

    def fire_emb(k, carry):
        pltpu.async_copy(emb_flat.at[idx_v.at[k]], gat_v.at[k], sem_e)
        return carry

    lax.fori_loop(0, EROWS, fire_emb, 0)

    def fire_fow(k, carry):
        pltpu.async_copy(fow_flat.at[idx_v.at[EROWS + k]], gat_v.at[EROWS + k],
                         sem_f)
        return carry

    lax.fori_loop(0, F, fire_fow, 0)

    def drain_emb(k, carry):
        pltpu.make_async_copy(emb_flat.at[idx_v.at[0]], gat_v.at[0],
                              sem_e).wait()
        return carry

    lax.fori_loop(0, EROWS, drain_emb, 0)

    def drain_fow(k, carry):
        pltpu.make_async_copy(fow_flat.at[idx_v.at[0]], gat_v.at[0],
                              sem_f).wait()
        return carry

    lax.fori_loop(0, F, drain_fow, 0)

    pltpu.sync_copy(gat_v, out.at[blk, :, pl.ds(off, LBLK)])


@functools.cache
def _sc_gather_fn():
    mesh = plsc.VectorSubcoreMesh(
        core_axis_name="c", subcore_axis_name="s",
        num_cores=NC, num_subcores=NS)
    return pl.kernel(
        _sc_body,
        out_type=jax.ShapeDtypeStruct((NBLK_H, GROWS, BBLK), jnp.float32),
        mesh=mesh,
        scratch_types=[
            pltpu.VMEM((GROWS, LBLK), jnp.int32),
            pltpu.VMEM((GROWS, LBLK), jnp.float32),
            pltpu.SemaphoreType.DMA,
            pltpu.SemaphoreType.DMA,
        ],
        compiler_params=pltpu.CompilerParams(use_tc_tiling_on_sc=False),
    )


def _tc_body(gat_ref, fvT_ref, bd_ref, bb_ref, hbd_ref,
             pp_ref, bias_ref, out_ref, bi_ref, log_ref):
    fv = fvT_ref[...]
    g = gat_ref[0]
    emb = g[0:EROWS].reshape(F, E, BBLK)
    fow = g[EROWS:EROWS + F]
    ev = emb * fv[:, None, :]

    off = 0
    for i in range(F - 1):
        n = F - 1 - i
        bi_ref[off:off + n] = ev[i + 1:F] * ev[i:i + 1]
        off += n
    bi_ref[P:P_PAD] = jnp.zeros((P_PAD - P, E, BBLK), jnp.float32)

    bd = bd_ref[...]
    bb = bb_ref[...]
    hbd = hbd_ref[...]
    for s in range(S):
        bi_s = bi_ref[s * 16:(s + 1) * 16].reshape(16 * E, BBLK)
        att = jnp.maximum(
            jnp.dot(bd, bi_s, preferred_element_type=jnp.float32) + bb, 0.0)
        log_ref[s * 16:(s + 1) * 16] = jnp.dot(
            hbd, att, preferred_element_type=jnp.float32)
    log_ref[P:P_PAD] = jnp.full((P_PAD - P, BBLK), -1e30, jnp.float32)

    logits = log_ref[...]
    m = jnp.max(logits, axis=0, keepdims=True)
    ex = jnp.exp(logits - m)
    z = jnp.sum(ex, axis=0, keepdims=True)
    score = ex / z

    aw = jnp.sum(score[:, None, :] * bi_ref[...], axis=0)
    awp = jnp.sum(aw * pp_ref[...], axis=0, keepdims=True)
    y1 = jnp.sum(fow * fv, axis=0, keepdims=True)
    y = bias_ref[...] + y1 + awp
    out_ref[...] = (1.0 / (1.0 + jnp.exp(-y)))[None]


def _tc_forward(gat, fvT, bd, bb, hbd, pp, bias_r):
    return pl.pallas_call(
        _tc_body,
        grid=(NBLK_H,),
        in_specs=[
            pl.BlockSpec((1, GROWS, BBLK), lambda i: (i, 0, 0)),
            pl.BlockSpec((F, BBLK), lambda i: (0, i)),
            pl.BlockSpec((16 * A, 16 * E), lambda i: (0, 0)),
            pl.BlockSpec((16 * A, 1), lambda i: (0, 0)),
            pl.BlockSpec((16, 16 * A), lambda i: (0, 0)),
            pl.BlockSpec((E, 1), lambda i: (0, 0)),
            pl.BlockSpec((1, 1), lambda i: (0, 0)),
        ],
        out_specs=pl.BlockSpec((1, 1, BBLK), lambda i: (i, 0, 0)),
        out_shape=jax.ShapeDtypeStruct((NBLK_H, 1, BBLK), jnp.float32),
        scratch_shapes=[
            pltpu.VMEM((P_PAD, E, BBLK), jnp.float32),
            pltpu.VMEM((P_PAD, BBLK), jnp.float32),
        ],
    )(gat, fvT, bd, bb, hbd, pp, bias_r)


def kernel(feat_index, feat_value, first_order_w, emb_table, bias,
           attention_w, attention_b, projection_h, projection_p):
    vT = feat_index.astype(jnp.int32).T
    v3 = vT.reshape(F, NBLK, BBLK).transpose(1, 0, 2)
    er = jnp.arange(E, dtype=jnp.int32)
    e_base = (er // 8) * (TCOLS * 1024) + (er % 8) * 128
    vhi = (v3 >> 7) * 1024 + (v3 & 127)
    emb_idx = (vhi[:, :, None, :] + e_base[None, None, :, None]
               ).reshape(NBLK, EROWS, BBLK)
    big_idx = jnp.concatenate(
        [emb_idx, vhi,
         jnp.zeros((NBLK, GROWS - EROWS - F, BBLK), jnp.int32)], axis=1)

    emb_flat = (jnp.pad(emb_table.T, ((0, 0), (0, 64)))
                .reshape(2, 8, TCOLS, 128).transpose(0, 2, 1, 3)
                .reshape(EMB_WORDS))
    fow_flat = (jnp.pad(first_order_w.T, ((0, 7), (0, 64)))
                .reshape(1, 8, TCOLS, 128).transpose(0, 2, 1, 3)
                .reshape(FOW_WORDS))

    fvT = feat_value.T
    eye = jnp.eye(16, dtype=jnp.float32)
    bd = jnp.kron(eye, attention_w.T)
    bb = jnp.tile(attention_b, 16)[:, None]
    hbd = jnp.kron(eye, projection_h[:, 0][None, :])
    bias_r = bias.reshape(1, 1)

    sc = _sc_gather_fn()
    outs = []
    for h in range(SPLIT):
        gat_h = sc(big_idx[h * NBLK_H:(h + 1) * NBLK_H], emb_flat, fow_flat)
        fvT_h = fvT[:, h * BH:(h + 1) * BH]
        outs.append(_tc_forward(gat_h, fvT_h, bd, bb, hbd,
                                projection_p, bias_r))
    return jnp.concatenate(outs, axis=0).reshape(BATCH)
